# Initial kernel scaffold; baseline (speedup 1.0000x reference)
import jax
import jax.numpy as jnp
from jax.experimental import pallas as pl


def kernel(x, edge_index, pos_edge_index, neg_edge_index, temp, W0, b0, W1, b1, Wm, bm, Wls, bls, Wd1, bd1, WdX, bdX, Wc1, bc1, WcA, bcA):
    raise NotImplementedError("write your pallas kernel here")


# speedup vs baseline: 1.2964x; 1.2964x over previous
import functools

import jax
import jax.numpy as jnp
from jax.experimental import pallas as pl

N = 10000
D_IN = 128
H0 = 512
H1 = 256
H2 = 128
H4 = 256
OUT = 128
CAT = 4

ROWS_BLK = 1000


def _head_body(mean_ref, ls_ref, noise_ref, wd1_ref, bd1_ref, wdx_ref, bdx_ref,
               wc1_ref, bc1_ref, wca_ref, bca_ref, x_ref, a_ref):
    mean = mean_ref[...]
    ls = ls_ref[...]
    noise = noise_ref[...]
    z = noise * jnp.exp(ls) + mean
    h = jnp.maximum(jnp.dot(z, wd1_ref[...], preferred_element_type=jnp.float32)
                    + bd1_ref[...], 0.0)
    x_ref[...] = jnp.maximum(
        jnp.dot(h, wdx_ref[...], preferred_element_type=jnp.float32) + bdx_ref[...], 0.0)
    c = jnp.maximum(jnp.dot(z, wc1_ref[...], preferred_element_type=jnp.float32)
                    + bc1_ref[...], 0.0)
    logits = jnp.dot(c, wca_ref[...], preferred_element_type=jnp.float32) + bca_ref[...]
    col = jax.lax.broadcasted_iota(jnp.int32, logits.shape, 1)
    valid = col < CAT
    logits = jnp.where(valid, logits, -jnp.inf)
    m = jnp.max(logits, axis=-1, keepdims=True)
    e = jnp.where(valid, jnp.exp(logits - m), 0.0)
    a_ref[...] = e / jnp.sum(e, axis=-1, keepdims=True)


def _head(mean, ls, noise, Wd1, bd1, WdX, bdX, Wc1, bc1, WcA, bcA):
    rows = mean.shape[0]
    grid = rows // ROWS_BLK
    row_spec = pl.BlockSpec((ROWS_BLK, H2), lambda i: (i, 0))
    full = lambda a: pl.BlockSpec(a.shape, lambda i: tuple(0 for _ in a.shape))
    WcA_p = jnp.zeros((H2 // 2, 128), jnp.float32).at[:, :CAT].set(WcA)
    bcA_p = jnp.zeros((1, 128), jnp.float32).at[0, :CAT].set(bcA)
    args = (mean, ls, noise, Wd1, bd1.reshape(1, -1), WdX, bdX.reshape(1, -1),
            Wc1, bc1.reshape(1, -1), WcA_p, bcA_p)
    out_x, out_a = pl.pallas_call(
        _head_body,
        grid=(grid,),
        in_specs=[row_spec, row_spec, row_spec] + [full(a) for a in args[3:]],
        out_specs=[pl.BlockSpec((ROWS_BLK, OUT), lambda i: (i, 0)),
                   pl.BlockSpec((ROWS_BLK, 128), lambda i: (i, 0))],
        out_shape=[jax.ShapeDtypeStruct((rows, OUT), jnp.float32),
                   jax.ShapeDtypeStruct((rows, 128), jnp.float32)],
    )(*args)
    return out_x, out_a[:, :CAT]


def kernel(x, edge_index, pos_edge_index, neg_edge_index, temp, W0, b0, W1, b1,
           Wm, bm, Wls, bls, Wd1, bd1, WdX, bdX, Wc1, bc1, WcA, bcA):
    src, dst = edge_index[0], edge_index[1]
    out_deg = jnp.zeros((N,), jnp.float32).at[src].add(1.0)
    in_deg = jnp.zeros((N,), jnp.float32).at[dst].add(1.0)
    ns = 1.0 / jnp.sqrt(jnp.clip(out_deg, 1.0))
    nd = 1.0 / jnp.sqrt(jnp.clip(in_deg, 1.0))

    v1 = x * ns[:, None]
    agg1 = jnp.zeros((N, D_IN), jnp.float32).at[dst].add(v1[src])
    h1 = jax.nn.relu((agg1 * nd[:, None]) @ W0 + b0)

    t = (h1 * ns[:, None]) @ W1
    agg2 = jnp.zeros((N, H1), jnp.float32).at[dst].add(t[src])
    h2 = jax.nn.relu(agg2 * nd[:, None] + b1)

    M = h2 @ Wm + 0.5 * bm
    LS = h2 @ Wls + 0.5 * bls
    pos_mean = M[pos_edge_index[0]] + M[pos_edge_index[1]]
    pos_logstd = LS[pos_edge_index[0]] + LS[pos_edge_index[1]]
    neg_mean = M[neg_edge_index[0]] + M[neg_edge_index[1]]
    neg_logstd = LS[neg_edge_index[0]] + LS[neg_edge_index[1]]

    kp, kn = jax.random.split(jax.random.key(42))
    noise_p = jax.random.normal(kp, pos_mean.shape, jnp.float32)
    noise_n = jax.random.normal(kn, neg_mean.shape, jnp.float32)

    head = functools.partial(
        _head, Wd1=Wd1, bd1=bd1, WdX=WdX, bdX=bdX, Wc1=Wc1, bc1=bc1, WcA=WcA, bcA=bcA)
    posX, posA = head(pos_mean, pos_logstd, noise_p)
    negX, negA = head(neg_mean, neg_logstd, noise_n)
    return (posA, negA, posX, negX, pos_mean, neg_mean, pos_logstd, neg_logstd)
